# baseline (speedup 1.0000x reference)
import functools

import jax
import jax.numpy as jnp
from jax import lax
from jax.experimental import pallas as pl
from jax.experimental.pallas import tpu as pltpu
from jax.experimental.pallas import tpu_sc as plsc

NUM_EMB = 1000000
EMB_DIM = 16
BATCH = 16384
HIST = 20
THD_NEG = -8.0
THD_POS = 7.0

NC = 2
NS = 16
NW = NC * NS

B_PER_W = BATCH // NW
CB = 64
CHUNKS = B_PER_W // CB
IDX_PER_CHUNK = CB * HIST
IDX_ROWS = IDX_PER_CHUNK // 128
IDX_PER_W = B_PER_W * HIST
ROUND_MAGIC = 12582912.0

SROWS = 2000
NSTRIPES = NUM_EMB // SROWS
SPITCH = SROWS + 1
MAX_STRIPES_PER_W = -(-NSTRIPES // NW)


def _tr_body(wt_hbm, wd_hbm, col_v, out_v, sem):
    wid = lax.axis_index("s") * NC + lax.axis_index("c")
    iota16 = lax.iota(jnp.int32, 16)

    def stripe_body(k, _):
        stripe = wid + k * NW

        @pl.when(stripe < NSTRIPES)
        def _():
            r0 = stripe * SROWS
            copies = [
                pltpu.async_copy(
                    wt_hbm.at[pl.ds(d, 1), pl.ds(r0, SROWS)],
                    col_v.at[pl.ds(d, 1), pl.ds(0, SROWS)],
                    sem,
                )
                for d in range(EMB_DIM)
            ]
            for cp in copies:
                cp.wait()

            def group_body(g, _):
                for j in range(16):
                    row = g * 16 + j
                    vals = plsc.load_gather(
                        col_v, [iota16, jnp.full((16,), row, jnp.int32)]
                    )
                    out_v[pl.ds(row * EMB_DIM, EMB_DIM)] = vals
                return 0

            lax.fori_loop(0, SROWS // 16, group_body, 0)
            pltpu.sync_copy(
                out_v, wd_hbm.at[pl.ds(stripe * (SROWS * EMB_DIM),
                                       SROWS * EMB_DIM)]
            )

        return 0

    lax.fori_loop(0, MAX_STRIPES_PER_W, stripe_body, 0)


def _bag_body(idx_hbm, w_hbm, s_hbm, out_hbm, idx_v, rows_v, out_v, s_v, sem):
    wid = lax.axis_index("s") * NC + lax.axis_index("c")

    pltpu.sync_copy(s_hbm, s_v)
    s_vec = s_v[...]

    pltpu.sync_copy(idx_hbm.at[pl.ds(wid * IDX_PER_W, IDX_PER_W)], idx_v)

    def chunk_body(c, _):
        base_out = (wid * B_PER_W + c * CB) * EMB_DIM

        copies = []
        for j in range(IDX_ROWS):
            copies.append(
                pltpu.async_copy(
                    w_hbm.at[idx_v.at[pl.ds((c * IDX_ROWS + j) * 128, 128)]],
                    rows_v.at[pl.ds(j * 128, 128)],
                    sem,
                )
            )
        for cp in copies:
            cp.wait()

        def bag_body(b, _):
            r0 = b * HIST
            acc = rows_v[r0, :]
            for h in range(1, HIST):
                acc = acc + rows_v[r0 + h, :]
            x = acc / s_vec
            x = jnp.minimum(jnp.maximum(x, THD_NEG), THD_POS)
            x = (x + ROUND_MAGIC) - ROUND_MAGIC
            out_v[pl.ds(b * EMB_DIM, EMB_DIM)] = x * s_vec
            return 0

        lax.fori_loop(0, CB, bag_body, 0)

        pltpu.sync_copy(out_v, out_hbm.at[pl.ds(base_out, CB * EMB_DIM)])
        return 0

    lax.fori_loop(0, CHUNKS, chunk_body, 0)


def kernel(indices, W, s):
    idx1d = indices.reshape(BATCH * HIST)
    s16 = jnp.broadcast_to(s, (16,)).astype(jnp.float32)
    wt = W.T

    mesh = plsc.VectorSubcoreMesh(core_axis_name="c", subcore_axis_name="s")
    params = pltpu.CompilerParams(use_tc_tiling_on_sc=False,
                                  needs_layout_passes=False)

    tr = functools.partial(
        pl.kernel,
        mesh=mesh,
        compiler_params=params,
        out_type=jax.ShapeDtypeStruct((NUM_EMB * EMB_DIM,), jnp.float32),
        scratch_types=[
            pltpu.VMEM((EMB_DIM, SPITCH), jnp.float32),
            pltpu.VMEM((SROWS * EMB_DIM,), jnp.float32),
            pltpu.SemaphoreType.DMA,
        ],
    )(_tr_body)
    wd = tr(wt)
    w2 = wd.reshape(NUM_EMB, EMB_DIM)

    bag = functools.partial(
        pl.kernel,
        mesh=mesh,
        compiler_params=params,
        out_type=jax.ShapeDtypeStruct((BATCH * EMB_DIM,), jnp.float32),
        scratch_types=[
            pltpu.VMEM((IDX_PER_W,), jnp.int32),
            pltpu.VMEM((IDX_PER_CHUNK, EMB_DIM), jnp.float32),
            pltpu.VMEM((CB * EMB_DIM,), jnp.float32),
            pltpu.VMEM((16,), jnp.float32),
            pltpu.SemaphoreType.DMA,
        ],
    )(_bag_body)
    out = bag(idx1d, w2, s16)
    return out.reshape(BATCH, EMB_DIM)


# speedup vs baseline: 2.9748x; 2.9748x over previous
import functools

import jax
import jax.numpy as jnp
from jax import lax
from jax.experimental import pallas as pl
from jax.experimental.pallas import tpu as pltpu
from jax.experimental.pallas import tpu_sc as plsc

NUM_EMB = 1000000
EMB_DIM = 16
BATCH = 16384
HIST = 20
THD_NEG = -8.0
THD_POS = 7.0

NC = 2
NS = 16
NW = NC * NS

B_PER_W = BATCH // NW
CB = 64
CHUNKS = B_PER_W // CB
IDX_PER_CHUNK = CB * HIST
IDX_ROWS = IDX_PER_CHUNK // 128
IDX_PER_W = B_PER_W * HIST
ROUND_MAGIC = 12582912.0

SR = 2048
NFULL = NUM_EMB // SR
TAIL = 512
NTAIL64 = 64
NSTRIPES = NFULL + 1
MAX_K = -(-NSTRIPES // NW)


def _tr_body(wt_hbm, wtail_hbm, wd_hbm, buf_v, out_v, sem):
    wid = lax.axis_index("s") * NC + lax.axis_index("c")
    iota16 = lax.iota(jnp.int32, 16)

    @pl.when(wid == NW - 1)
    def _():
        pltpu.sync_copy(wtail_hbm, out_v.at[pl.ds(0, NTAIL64 * EMB_DIM)])
        pltpu.sync_copy(
            out_v.at[pl.ds(0, NTAIL64 * EMB_DIM)],
            wd_hbm.at[pl.ds((NUM_EMB - NTAIL64) * EMB_DIM,
                            NTAIL64 * EMB_DIM)],
        )

    def do_stripe(stripe, n):
        c0 = stripe * SR
        lo = pltpu.async_copy(
            wt_hbm.at[pl.ds(0, 8), pl.ds(c0, n)],
            buf_v.at[pl.ds(0, 8), pl.ds(0, n)],
            sem,
        )
        hi = pltpu.async_copy(
            wt_hbm.at[pl.ds(8, 8), pl.ds(c0, n)],
            buf_v.at[pl.ds(8, 8), pl.ds(0, n)],
            sem,
        )
        lo.wait()
        hi.wait()

        def group_body(g, cvec):
            for j in range(16):
                vals = plsc.load_gather(buf_v, [iota16, cvec])
                out_v[pl.ds((g * 16 + j) * EMB_DIM, EMB_DIM)] = vals
                cvec = cvec + 1
            return cvec

        lax.fori_loop(0, n // 16, group_body, jnp.zeros((16,), jnp.int32))
        pltpu.sync_copy(
            out_v.at[pl.ds(0, n * EMB_DIM)],
            wd_hbm.at[pl.ds(c0 * EMB_DIM, n * EMB_DIM)],
        )

    def stripe_body(k, _):
        stripe = wid + k * NW

        @pl.when(stripe < NFULL)
        def _():
            do_stripe(stripe, SR)

        @pl.when(stripe == NFULL)
        def _():
            do_stripe(stripe, TAIL)

        return 0

    lax.fori_loop(0, MAX_K, stripe_body, 0)


def _bag_body(idx_hbm, w_hbm, s_hbm, out_hbm, idx_v, rows_v, out_v, s_v, sem):
    wid = lax.axis_index("s") * NC + lax.axis_index("c")

    pltpu.sync_copy(s_hbm, s_v)
    s_vec = s_v[...]

    pltpu.sync_copy(idx_hbm.at[pl.ds(wid * IDX_PER_W, IDX_PER_W)], idx_v)

    def chunk_body(c, _):
        base_out = (wid * B_PER_W + c * CB) * EMB_DIM

        copies = []
        for j in range(IDX_ROWS):
            copies.append(
                pltpu.async_copy(
                    w_hbm.at[idx_v.at[pl.ds((c * IDX_ROWS + j) * 128, 128)]],
                    rows_v.at[pl.ds(j * 128, 128)],
                    sem,
                )
            )
        for cp in copies:
            cp.wait()

        def bag_body(b, _):
            r0 = b * HIST
            acc = rows_v[r0, :]
            for h in range(1, HIST):
                acc = acc + rows_v[r0 + h, :]
            x = acc / s_vec
            x = jnp.minimum(jnp.maximum(x, THD_NEG), THD_POS)
            x = (x + ROUND_MAGIC) - ROUND_MAGIC
            out_v[pl.ds(b * EMB_DIM, EMB_DIM)] = x * s_vec
            return 0

        lax.fori_loop(0, CB, bag_body, 0)

        pltpu.sync_copy(out_v, out_hbm.at[pl.ds(base_out, CB * EMB_DIM)])
        return 0

    lax.fori_loop(0, CHUNKS, chunk_body, 0)


def kernel(indices, W, s):
    idx1d = indices.reshape(BATCH * HIST)
    s16 = jnp.broadcast_to(s, (16,)).astype(jnp.float32)
    wt = W.T
    wtail = W[NUM_EMB - NTAIL64:, :].reshape(NTAIL64 * EMB_DIM)

    mesh = plsc.VectorSubcoreMesh(core_axis_name="c", subcore_axis_name="s")

    tr = functools.partial(
        pl.kernel,
        mesh=mesh,
        compiler_params=pltpu.CompilerParams(use_tc_tiling_on_sc=True,
                                             needs_layout_passes=False),
        out_type=jax.ShapeDtypeStruct((NUM_EMB * EMB_DIM,), jnp.float32),
        scratch_types=[
            pltpu.VMEM((EMB_DIM, SR), jnp.float32),
            pltpu.VMEM((SR * EMB_DIM,), jnp.float32),
            pltpu.SemaphoreType.DMA,
        ],
    )(_tr_body)
    wd = tr(wt, wtail)
    w2 = wd.reshape(NUM_EMB, EMB_DIM)

    bag = functools.partial(
        pl.kernel,
        mesh=mesh,
        compiler_params=pltpu.CompilerParams(use_tc_tiling_on_sc=False,
                                             needs_layout_passes=False),
        out_type=jax.ShapeDtypeStruct((BATCH * EMB_DIM,), jnp.float32),
        scratch_types=[
            pltpu.VMEM((IDX_PER_W,), jnp.int32),
            pltpu.VMEM((IDX_PER_CHUNK, EMB_DIM), jnp.float32),
            pltpu.VMEM((CB * EMB_DIM,), jnp.float32),
            pltpu.VMEM((16,), jnp.float32),
            pltpu.SemaphoreType.DMA,
        ],
    )(_bag_body)
    out = bag(idx1d, w2, s16)
    return out.reshape(BATCH, EMB_DIM)


# speedup vs baseline: 5.5504x; 1.8658x over previous
import functools

import jax
import jax.numpy as jnp
from jax import lax
from jax.experimental import pallas as pl
from jax.experimental.pallas import tpu as pltpu
from jax.experimental.pallas import tpu_sc as plsc

NUM_EMB = 1000000
EMB_DIM = 16
BATCH = 16384
HIST = 20
THD_NEG = -8.0
THD_POS = 7.0

NC = 2
NS = 16
NW = NC * NS

B_PER_W = BATCH // NW
CB = 64
CHUNKS = B_PER_W // CB
IDX_PER_CHUNK = CB * HIST
IDX_ROWS = IDX_PER_CHUNK // 128
IDX_PER_W = B_PER_W * HIST
ROUND_MAGIC = 12582912.0

SR = 2048
NFULL = NUM_EMB // SR
TAIL = 512
NTAIL64 = 64
NSTRIPES = NFULL + 1
MAX_K = -(-NSTRIPES // NW)


def _tr_body(wt_hbm, wtail_hbm, wd_hbm, buf_v, out_v, sem):
    wid = lax.axis_index("s") * NC + lax.axis_index("c")
    iota16 = lax.iota(jnp.int32, 16)

    @pl.when(wid == NW - 1)
    def _():
        pltpu.sync_copy(wtail_hbm, out_v.at[pl.ds(0, NTAIL64 * EMB_DIM)])
        pltpu.sync_copy(
            out_v.at[pl.ds(0, NTAIL64 * EMB_DIM)],
            wd_hbm.at[pl.ds((NUM_EMB - NTAIL64) * EMB_DIM,
                            NTAIL64 * EMB_DIM)],
        )

    def do_stripe(stripe, n):
        c0 = stripe * SR
        lo = pltpu.async_copy(
            wt_hbm.at[pl.ds(0, 8), pl.ds(c0, n)],
            buf_v.at[pl.ds(0, 8), pl.ds(0, n)],
            sem,
        )
        hi = pltpu.async_copy(
            wt_hbm.at[pl.ds(8, 8), pl.ds(c0, n)],
            buf_v.at[pl.ds(8, 8), pl.ds(0, n)],
            sem,
        )
        lo.wait()
        hi.wait()

        dvecs = [jnp.bitwise_and(iota16 + k, 15) for k in range(16)]

        def block_body(g, cvec):
            pvec = cvec * EMB_DIM
            for k in range(16):
                vals = plsc.load_gather(buf_v, [dvecs[k], cvec])
                plsc.store_scatter(out_v, [pvec + dvecs[k]], vals)
            return cvec + 16

        lax.fori_loop(0, n // 16, block_body, iota16)
        pl.delay(100)
        pltpu.sync_copy(
            out_v.at[pl.ds(0, n * EMB_DIM)],
            wd_hbm.at[pl.ds(c0 * EMB_DIM, n * EMB_DIM)],
        )

    def stripe_body(k, _):
        stripe = wid + k * NW

        @pl.when(stripe < NFULL)
        def _():
            do_stripe(stripe, SR)

        @pl.when(stripe == NFULL)
        def _():
            do_stripe(stripe, TAIL)

        return 0

    lax.fori_loop(0, MAX_K, stripe_body, 0)


def _bag_body(idx_hbm, w_hbm, s_hbm, out_hbm, idx_v, rows_v, out_v, s_v, sem):
    wid = lax.axis_index("s") * NC + lax.axis_index("c")

    pltpu.sync_copy(s_hbm, s_v)
    s_vec = s_v[...]

    pltpu.sync_copy(idx_hbm.at[pl.ds(wid * IDX_PER_W, IDX_PER_W)], idx_v)

    def chunk_body(c, _):
        base_out = (wid * B_PER_W + c * CB) * EMB_DIM

        copies = []
        for j in range(IDX_ROWS):
            copies.append(
                pltpu.async_copy(
                    w_hbm.at[idx_v.at[pl.ds((c * IDX_ROWS + j) * 128, 128)]],
                    rows_v.at[pl.ds(j * 128, 128)],
                    sem,
                )
            )
        for cp in copies:
            cp.wait()

        def bag_body(b, _):
            r0 = b * HIST
            acc = rows_v[r0, :]
            for h in range(1, HIST):
                acc = acc + rows_v[r0 + h, :]
            x = acc / s_vec
            x = jnp.minimum(jnp.maximum(x, THD_NEG), THD_POS)
            x = (x + ROUND_MAGIC) - ROUND_MAGIC
            out_v[pl.ds(b * EMB_DIM, EMB_DIM)] = x * s_vec
            return 0

        lax.fori_loop(0, CB, bag_body, 0)

        pltpu.sync_copy(out_v, out_hbm.at[pl.ds(base_out, CB * EMB_DIM)])
        return 0

    lax.fori_loop(0, CHUNKS, chunk_body, 0)


def kernel(indices, W, s):
    idx1d = indices.reshape(BATCH * HIST)
    s16 = jnp.broadcast_to(s, (16,)).astype(jnp.float32)
    wt = W.T
    wtail = W[NUM_EMB - NTAIL64:, :].reshape(NTAIL64 * EMB_DIM)

    mesh = plsc.VectorSubcoreMesh(core_axis_name="c", subcore_axis_name="s")

    tr = functools.partial(
        pl.kernel,
        mesh=mesh,
        compiler_params=pltpu.CompilerParams(use_tc_tiling_on_sc=True,
                                             needs_layout_passes=False),
        out_type=jax.ShapeDtypeStruct((NUM_EMB * EMB_DIM,), jnp.float32),
        scratch_types=[
            pltpu.VMEM((EMB_DIM, SR), jnp.float32),
            pltpu.VMEM((SR * EMB_DIM,), jnp.float32),
            pltpu.SemaphoreType.DMA,
        ],
    )(_tr_body)
    wd = tr(wt, wtail)
    w2 = wd.reshape(NUM_EMB, EMB_DIM)

    bag = functools.partial(
        pl.kernel,
        mesh=mesh,
        compiler_params=pltpu.CompilerParams(use_tc_tiling_on_sc=False,
                                             needs_layout_passes=False),
        out_type=jax.ShapeDtypeStruct((BATCH * EMB_DIM,), jnp.float32),
        scratch_types=[
            pltpu.VMEM((IDX_PER_W,), jnp.int32),
            pltpu.VMEM((IDX_PER_CHUNK, EMB_DIM), jnp.float32),
            pltpu.VMEM((CB * EMB_DIM,), jnp.float32),
            pltpu.VMEM((16,), jnp.float32),
            pltpu.SemaphoreType.DMA,
        ],
    )(_bag_body)
    out = bag(idx1d, w2, s16)
    return out.reshape(BATCH, EMB_DIM)
